# baseline (speedup 1.0000x reference)
import functools

import jax
import jax.numpy as jnp
from jax import lax
from jax.experimental import pallas as pl
from jax.experimental.pallas import tpu as pltpu
from jax.experimental.pallas import tpu_sc as plsc

B = 4096
S = 200
D = 64
NC = 2
NS = 16
NW = NC * NS
RPW = B // NW
C0, C1 = 128, S - 128
UNROLL = 8
INV_S = 1.0 / S

_mesh = plsc.VectorSubcoreMesh(core_axis_name="c", subcore_axis_name="s")


@functools.partial(
    pl.kernel,
    out_type=jax.ShapeDtypeStruct((B, D), jnp.float32),
    mesh=_mesh,
    scratch_types=[
        pltpu.VMEM((RPW * S,), jnp.int32),
        pltpu.VMEM((2, S, D), jnp.float32),
        pltpu.VMEM((RPW, D), jnp.float32),
        pltpu.SemaphoreType.DMA,
        pltpu.SemaphoreType.DMA,
    ],
    compiler_params=pltpu.CompilerParams(use_tc_tiling_on_sc=False),
)
def _embed_mean(x_hbm, table_hbm, out_hbm, idx_v, rows_v, out_v, sem0, sem1):
    wid = lax.axis_index("s") * NC + lax.axis_index("c")
    base = wid * RPW

    pltpu.sync_copy(x_hbm.at[pl.ds(base * S, RPW * S)], idx_v)

    def _copies(r, buf, sem):
        off = pl.multiple_of(r * S, 8)
        c0 = pltpu.make_async_copy(
            table_hbm.at[idx_v.at[pl.ds(off, C0)]],
            rows_v.at[buf].at[pl.ds(0, C0)], sem)
        c1 = pltpu.make_async_copy(
            table_hbm.at[idx_v.at[pl.ds(off + C0, C1)]],
            rows_v.at[buf].at[pl.ds(C0, C1)], sem)
        return c0, c1

    def fire(r, buf, sem):
        for c in _copies(r, buf, sem):
            c.start()

    def drain(r, buf, sem):
        for c in _copies(r, buf, sem):
            c.wait()

    def accum_store(r, buf):
        rows = rows_v.at[buf]
        zero = jnp.zeros((16,), jnp.float32)

        def body(i, acc):
            a = list(acc)
            s0 = i * UNROLL
            for u in range(UNROLL):
                for c in range(4):
                    k = (u % 2) * 4 + c
                    a[k] = a[k] + rows[s0 + u, pl.ds(c * 16, 16)]
            return tuple(a)

        acc = lax.fori_loop(0, S // UNROLL, body, (zero,) * 8)
        for c in range(4):
            out_v[r, pl.ds(c * 16, 16)] = (acc[c] + acc[4 + c]) * INV_S

    fire(0, 0, sem0)

    @pl.loop(0, RPW, step=2)
    def _(g):
        fire(g + 1, 1, sem1)
        drain(g, 0, sem0)
        accum_store(g, 0)

        @pl.when(g + 2 < RPW)
        def _():
            fire(g + 2, 0, sem0)

        drain(g + 1, 1, sem1)
        accum_store(g + 1, 1)

    pltpu.sync_copy(out_v, out_hbm.at[pl.ds(base, RPW)])


def kernel(x, table):
    return _embed_mean(x.reshape(-1), table)
